# initial kernel scaffold (unmeasured)
import jax
import jax.numpy as jnp
from jax import lax
from jax.experimental import pallas as pl
from jax.experimental.pallas import tpu as pltpu

N_DEV = 4
SQ = 512
SKV = 2048
D = 1024
HQ_TOT = 32
HQ = 8
DH = 128
SCALE = 0.08838834764831843


def kernel(x, Wq, Wo, K_ext, V_ext):
    x2 = x.reshape(SQ, D)
    K = K_ext.reshape(SKV, HQ_TOT, DH)
    V = V_ext.reshape(SKV, HQ_TOT, DH)

    def body(x_ref, wq_ref, wo_ref, k_hbm, v_hbm, out_ref,
             k_vmem, v_vmem, attn_ref, comm_ref,
             kv_sems, send_sems, recv_sems):
        my = lax.axis_index("i")

        kcp = pltpu.make_async_copy(
            k_hbm.at[:, pl.ds(my * HQ, HQ), :], k_vmem, kv_sems.at[0])
        vcp = pltpu.make_async_copy(
            v_hbm.at[:, pl.ds(my * HQ, HQ), :], v_vmem, kv_sems.at[1])
        kcp.start()
        vcp.start()

        barrier = pltpu.get_barrier_semaphore()
        for k in (1, 2, 3):
            pl.semaphore_signal(
                barrier, inc=1,
                device_id=((my + k) % N_DEV,),
                device_id_type=pl.DeviceIdType.MESH,
            )
        pl.semaphore_wait(barrier, N_DEV - 1)

        xb = x_ref[...].astype(jnp.bfloat16)
        wqb = wq_ref[...].astype(jnp.bfloat16)
        q = lax.dot(xb, wqb, preferred_element_type=jnp.float32)
        q = (q * SCALE).astype(jnp.bfloat16)

        kcp.wait()
        vcp.wait()

        for h in range(HQ):
            qh = q[:, h * DH:(h + 1) * DH]
            kh = k_vmem[:, h, :].astype(jnp.bfloat16)
            s = lax.dot_general(
                qh, kh, (((1,), (1,)), ((), ())),
                preferred_element_type=jnp.float32)
            m = jnp.max(s, axis=1, keepdims=True)
            p = jnp.exp(s - m)
            l = jnp.sum(p, axis=1, keepdims=True)
            vh = v_vmem[:, h, :].astype(jnp.bfloat16)
            o = lax.dot(p.astype(jnp.bfloat16), vh,
                        preferred_element_type=jnp.float32)
            attn_ref[:, h * DH:(h + 1) * DH] = (o / l).astype(jnp.bfloat16)

        partial = lax.dot(attn_ref[...], wo_ref[...].astype(jnp.bfloat16),
                          preferred_element_type=jnp.float32)
        comm_ref[0, :, :] = partial.astype(jnp.bfloat16)

        rdmas = []
        for k in (1, 2, 3):
            r = pltpu.make_async_remote_copy(
                src_ref=comm_ref.at[0],
                dst_ref=comm_ref.at[N_DEV - k],
                send_sem=send_sems.at[k - 1],
                recv_sem=recv_sems.at[N_DEV - k],
                device_id=((my + k) % N_DEV,),
                device_id_type=pl.DeviceIdType.MESH,
            )
            r.start()
            rdmas.append(r)
        for r in rdmas:
            r.wait_recv()

        out_ref[...] = (comm_ref[0, :, :].astype(jnp.float32)
                        + comm_ref[1, :, :].astype(jnp.float32)
                        + comm_ref[2, :, :].astype(jnp.float32)
                        + comm_ref[3, :, :].astype(jnp.float32))

        for r in rdmas:
            r.wait_send()

    out = pl.pallas_call(
        body,
        out_shape=jax.ShapeDtypeStruct((SQ, D), jnp.float32),
        in_specs=[
            pl.BlockSpec(memory_space=pltpu.VMEM),
            pl.BlockSpec(memory_space=pltpu.VMEM),
            pl.BlockSpec(memory_space=pltpu.VMEM),
            pl.BlockSpec(memory_space=pltpu.ANY),
            pl.BlockSpec(memory_space=pltpu.ANY),
        ],
        out_specs=pl.BlockSpec(memory_space=pltpu.VMEM),
        scratch_shapes=[
            pltpu.VMEM((SKV, HQ, DH), jnp.float32),
            pltpu.VMEM((SKV, HQ, DH), jnp.float32),
            pltpu.VMEM((SQ, HQ * DH), jnp.bfloat16),
            pltpu.VMEM((N_DEV, SQ, D), jnp.bfloat16),
            pltpu.SemaphoreType.DMA((2,)),
            pltpu.SemaphoreType.DMA((3,)),
            pltpu.SemaphoreType.DMA((N_DEV,)),
        ],
        compiler_params=pltpu.CompilerParams(collective_id=0),
    )(x2, Wq, Wo, K, V)

    return out.reshape(1, SQ, D)


# baseline (device time: 69516 ns/iter reference)
import jax
import jax.numpy as jnp
from jax import lax
from jax.experimental import pallas as pl
from jax.experimental.pallas import tpu as pltpu

N_DEV = 4
SQ = 512
SKV = 2048
D = 1024
HQ_TOT = 32
HQ = 8
DH = 128
SCALE = 0.08838834764831843


def kernel(x, Wq, Wo, K_ext, V_ext):
    x2 = x.reshape(SQ, D)
    K = K_ext.reshape(SKV, HQ_TOT, DH)
    V = V_ext.reshape(SKV, HQ_TOT, DH)

    def body(x_ref, wq_ref, wo_ref, k_hbm, v_hbm, out_ref,
             k_vmem, v_vmem, attn_ref, comm_ref,
             kv_sems, send_sems, recv_sems):
        my = lax.axis_index("i")

        kcp = pltpu.make_async_copy(
            k_hbm.at[:, pl.ds(my * HQ, HQ), :], k_vmem, kv_sems.at[0])
        vcp = pltpu.make_async_copy(
            v_hbm.at[:, pl.ds(my * HQ, HQ), :], v_vmem, kv_sems.at[1])
        kcp.start()
        vcp.start()

        barrier = pltpu.get_barrier_semaphore()
        for k in (1, 2, 3):
            pl.semaphore_signal(
                barrier, inc=1,
                device_id=((my + k) % N_DEV,),
                device_id_type=pl.DeviceIdType.MESH,
            )
        pl.semaphore_wait(barrier, N_DEV - 1)

        xb = x_ref[...].astype(jnp.bfloat16)
        wqb = wq_ref[...].astype(jnp.bfloat16)
        q = lax.dot(xb, wqb, preferred_element_type=jnp.float32)
        q = (q * SCALE).astype(jnp.bfloat16)

        kcp.wait()
        vcp.wait()

        for h in range(HQ):
            qh = q[:, h * DH:(h + 1) * DH]
            kh = k_vmem[:, h, :].astype(jnp.bfloat16)
            s = lax.dot_general(
                qh, kh, (((1,), (1,)), ((), ())),
                preferred_element_type=jnp.float32)
            m = jnp.max(s, axis=1, keepdims=True)
            p = jnp.exp(s - m)
            l = jnp.sum(p, axis=1, keepdims=True)
            vh = v_vmem[:, h, :].astype(jnp.bfloat16)
            o = lax.dot(p.astype(jnp.bfloat16), vh,
                        preferred_element_type=jnp.float32)
            attn_ref[:, h * DH:(h + 1) * DH] = (o / l).astype(jnp.bfloat16)

        partial = lax.dot(attn_ref[...], wo_ref[...].astype(jnp.bfloat16),
                          preferred_element_type=jnp.float32)
        comm_ref[0, :, :] = partial.astype(jnp.bfloat16)

        rdmas = []
        for k in (1, 2, 3):
            r = pltpu.make_async_remote_copy(
                src_ref=comm_ref.at[0],
                dst_ref=comm_ref.at[N_DEV - k],
                send_sem=send_sems.at[k - 1],
                recv_sem=recv_sems.at[N_DEV - k],
                device_id=((my + k) % N_DEV,),
                device_id_type=pl.DeviceIdType.MESH,
            )
            r.start()
            rdmas.append(r)
        for r in rdmas:
            r.wait_recv()

        out_ref[...] = (comm_ref[0, :, :].astype(jnp.float32)
                        + comm_ref[1, :, :].astype(jnp.float32)
                        + comm_ref[2, :, :].astype(jnp.float32)
                        + comm_ref[3, :, :].astype(jnp.float32))

        for r in rdmas:
            r.wait_send()

    out = pl.pallas_call(
        body,
        out_shape=jax.ShapeDtypeStruct((SQ, D), jnp.float32),
        in_specs=[
            pl.BlockSpec(memory_space=pltpu.VMEM),
            pl.BlockSpec(memory_space=pltpu.VMEM),
            pl.BlockSpec(memory_space=pltpu.VMEM),
            pl.BlockSpec(memory_space=pl.ANY),
            pl.BlockSpec(memory_space=pl.ANY),
        ],
        out_specs=pl.BlockSpec(memory_space=pltpu.VMEM),
        scratch_shapes=[
            pltpu.VMEM((SKV, HQ, DH), jnp.float32),
            pltpu.VMEM((SKV, HQ, DH), jnp.float32),
            pltpu.VMEM((SQ, HQ * DH), jnp.bfloat16),
            pltpu.VMEM((N_DEV, SQ, D), jnp.bfloat16),
            pltpu.SemaphoreType.DMA((2,)),
            pltpu.SemaphoreType.DMA((3,)),
            pltpu.SemaphoreType.DMA((N_DEV,)),
        ],
        compiler_params=pltpu.CompilerParams(
            collective_id=0,
            vmem_limit_bytes=100 * 1024 * 1024,
        ),
    )(x2, Wq, Wo, K, V)

    return out.reshape(1, SQ, D)


# device time: 58269 ns/iter; 1.1930x vs baseline; 1.1930x over previous
import jax
import jax.numpy as jnp
from jax import lax
from jax.experimental import pallas as pl
from jax.experimental.pallas import tpu as pltpu

N_DEV = 4
SQ = 512
SKV = 2048
D = 1024
HQ_TOT = 32
HQ = 8
DH = 128
SCALE = 0.08838834764831843
NC = 4
CH = SQ // NC


def kernel(x, Wq, Wo, K_ext, V_ext):
    x2 = x.reshape(SQ, D)
    K = K_ext.reshape(SKV, HQ_TOT, DH)
    V = V_ext.reshape(SKV, HQ_TOT, DH)

    def body(x_ref, wq_ref, wo_ref, k_hbm, v_hbm, out_ref,
             k_refs, v_refs, q_ref, wob_ref, comm_ref,
             kv_sems, send_sems, recv_sems):
        my = lax.axis_index("i")

        kv_copies = []
        for h in range(HQ):
            kc = pltpu.make_async_copy(
                k_hbm.at[:, my * HQ + h, :], k_refs.at[h], kv_sems.at[h])
            vc = pltpu.make_async_copy(
                v_hbm.at[:, my * HQ + h, :], v_refs.at[h], kv_sems.at[HQ + h])
            kc.start()
            vc.start()
            kv_copies += [kc, vc]

        barrier = pltpu.get_barrier_semaphore()
        for k in (1, 2, 3):
            pl.semaphore_signal(
                barrier, inc=1,
                device_id=((my + k) % N_DEV,),
                device_id_type=pl.DeviceIdType.MESH,
            )
        pl.semaphore_wait(barrier, N_DEV - 1)

        xb = x_ref[...].astype(jnp.bfloat16)
        wqb = wq_ref[...].astype(jnp.bfloat16)
        q = lax.dot(xb, wqb, preferred_element_type=jnp.float32)
        q = (q * SCALE).astype(jnp.bfloat16)
        for h in range(HQ):
            q_ref[h] = q[:, h * DH:(h + 1) * DH]
            wob_ref[h] = wo_ref[h * DH:(h + 1) * DH, :].astype(jnp.bfloat16)

        for c in kv_copies:
            c.wait()

        rdmas = []
        for c in range(NC):
            def head_step(h, acc, c=c):
                qh = q_ref[h, c * CH:(c + 1) * CH, :]
                kh = k_refs[h].astype(jnp.bfloat16)
                s = lax.dot_general(
                    qh, kh, (((1,), (1,)), ((), ())),
                    preferred_element_type=jnp.float32)
                m = jnp.max(s, axis=1, keepdims=True)
                p = jnp.exp(s - m)
                l = jnp.sum(p, axis=1, keepdims=True)
                o = lax.dot(p.astype(jnp.bfloat16),
                            v_refs[h].astype(jnp.bfloat16),
                            preferred_element_type=jnp.float32)
                ob = (o / l).astype(jnp.bfloat16)
                return acc + lax.dot(ob, wob_ref[h],
                                     preferred_element_type=jnp.float32)

            pc = lax.fori_loop(
                0, HQ, head_step, jnp.zeros((CH, D), jnp.float32))
            comm_ref[0, c, :, :] = pc.astype(jnp.bfloat16)
            for k in (1, 2, 3):
                r = pltpu.make_async_remote_copy(
                    src_ref=comm_ref.at[0, c],
                    dst_ref=comm_ref.at[N_DEV - k, c],
                    send_sem=send_sems.at[c * 3 + k - 1],
                    recv_sem=recv_sems.at[(N_DEV - k) * NC + c],
                    device_id=((my + k) % N_DEV,),
                    device_id_type=pl.DeviceIdType.MESH,
                )
                r.start()
                rdmas.append(r)

        for c in range(NC):
            rows = pl.ds(c * CH, CH)
            for k in (1, 2, 3):
                rdmas[c * 3 + (k - 1)].wait_recv()
            out_ref[rows, :] = (
                comm_ref[0, c, :, :].astype(jnp.float32)
                + comm_ref[1, c, :, :].astype(jnp.float32)
                + comm_ref[2, c, :, :].astype(jnp.float32)
                + comm_ref[3, c, :, :].astype(jnp.float32))

        for r in rdmas:
            r.wait_send()

    out = pl.pallas_call(
        body,
        out_shape=jax.ShapeDtypeStruct((SQ, D), jnp.float32),
        in_specs=[
            pl.BlockSpec(memory_space=pltpu.VMEM),
            pl.BlockSpec(memory_space=pltpu.VMEM),
            pl.BlockSpec(memory_space=pltpu.VMEM),
            pl.BlockSpec(memory_space=pl.ANY),
            pl.BlockSpec(memory_space=pl.ANY),
        ],
        out_specs=pl.BlockSpec(memory_space=pltpu.VMEM),
        scratch_shapes=[
            pltpu.VMEM((HQ, SKV, DH), jnp.float32),
            pltpu.VMEM((HQ, SKV, DH), jnp.float32),
            pltpu.VMEM((HQ, SQ, DH), jnp.bfloat16),
            pltpu.VMEM((HQ, DH, D), jnp.bfloat16),
            pltpu.VMEM((N_DEV, NC, CH, D), jnp.bfloat16),
            pltpu.SemaphoreType.DMA((2 * HQ,)),
            pltpu.SemaphoreType.DMA((3 * NC,)),
            pltpu.SemaphoreType.DMA((N_DEV * NC,)),
        ],
        compiler_params=pltpu.CompilerParams(
            collective_id=0,
            vmem_limit_bytes=63 * 1024 * 1024,
        ),
    )(x2, Wq, Wo, K, V)

    return out.reshape(1, SQ, D)
